# speedup vs baseline: 2.2308x; 2.2308x over previous
import jax
import jax.numpy as jnp
from jax.experimental import pallas as pl
from jax.experimental.pallas import tpu as pltpu

STATE_DIM = 128
ACTION_DIM = 32
HIDDEN = 256
E = 8
OUT_DIM = STATE_DIM + 1


def _fused_kernel(x_ref, state_ref, idx_ref, W1_ref, b1_ref, W2_ref, b2_ref,
                  W3_ref, b3_ref, ns_ref, rw_ref):
    x = x_ref[:]
    idx = idx_ref[:]
    acc = jnp.zeros((x.shape[0], OUT_DIM), dtype=jnp.float32)
    for e in range(E):
        h1 = jnp.maximum(
            jnp.dot(x, W1_ref[e], preferred_element_type=jnp.float32) + b1_ref[e], 0.0)
        h2 = jnp.maximum(
            jnp.dot(h1, W2_ref[e], preferred_element_type=jnp.float32) + b2_ref[e], 0.0)
        o = jnp.dot(h2, W3_ref[e], preferred_element_type=jnp.float32) + b3_ref[e]
        acc = jnp.where(idx == e, o, acc)
    ns_ref[:] = state_ref[:] + acc[:, :STATE_DIM]
    rw_ref[:] = acc[:, STATE_DIM:]


@jax.jit
def kernel(state, action, W1, b1, W2, b2, W3, b3, idx):
    B = state.shape[0]
    in_dim = STATE_DIM + ACTION_DIM
    x = jnp.concatenate([state, action], axis=-1)
    idx2 = idx.astype(jnp.int32).reshape(B, 1)

    T = 512
    grid = (B // T,)

    def full(shape):
        return pl.BlockSpec(shape, lambda i: (0,) * len(shape))

    ns, rw = pl.pallas_call(
        _fused_kernel,
        grid=grid,
        in_specs=[
            pl.BlockSpec((T, in_dim), lambda i: (i, 0)),
            pl.BlockSpec((T, STATE_DIM), lambda i: (i, 0)),
            pl.BlockSpec((T, 1), lambda i: (i, 0)),
            full((E, in_dim, HIDDEN)),
            full((E, HIDDEN)),
            full((E, HIDDEN, HIDDEN)),
            full((E, HIDDEN)),
            full((E, HIDDEN, OUT_DIM)),
            full((E, OUT_DIM)),
        ],
        out_specs=[
            pl.BlockSpec((T, STATE_DIM), lambda i: (i, 0)),
            pl.BlockSpec((T, 1), lambda i: (i, 0)),
        ],
        out_shape=[
            jax.ShapeDtypeStruct((B, STATE_DIM), jnp.float32),
            jax.ShapeDtypeStruct((B, 1), jnp.float32),
        ],
        compiler_params=pltpu.CompilerParams(
            dimension_semantics=("parallel",)),
    )(x, state, idx2, W1, b1, W2, b2, W3, b3)
    return (ns, rw)


# speedup vs baseline: 2.2969x; 1.0296x over previous
import jax
import jax.numpy as jnp
from jax.experimental import pallas as pl
from jax.experimental.pallas import tpu as pltpu

STATE_DIM = 128
ACTION_DIM = 32
HIDDEN = 256
E = 8
OUT_DIM = STATE_DIM + 1


def _fused_kernel(x_ref, state_ref, idx_ref, W1_ref, b1_ref, W2_ref, b2_ref,
                  W3_ref, b3_ref, ns_ref, rw_ref):
    x = x_ref[:]
    idx = idx_ref[:]
    acc = jnp.zeros((x.shape[0], OUT_DIM), dtype=jnp.float32)
    for e in range(E):
        h1 = jnp.maximum(
            jnp.dot(x, W1_ref[e], preferred_element_type=jnp.float32) + b1_ref[e], 0.0)
        h2 = jnp.maximum(
            jnp.dot(h1.astype(jnp.bfloat16), W2_ref[e],
                    preferred_element_type=jnp.float32) + b2_ref[e], 0.0)
        o = jnp.dot(h2.astype(jnp.bfloat16), W3_ref[e],
                    preferred_element_type=jnp.float32) + b3_ref[e]
        acc = jnp.where(idx == e, o, acc)
    ns_ref[:] = state_ref[:] + acc[:, :STATE_DIM]
    rw_ref[:] = acc[:, STATE_DIM:]


@jax.jit
def kernel(state, action, W1, b1, W2, b2, W3, b3, idx):
    B = state.shape[0]
    in_dim = STATE_DIM + ACTION_DIM
    x = jnp.concatenate([state, action], axis=-1).astype(jnp.bfloat16)
    W1 = W1.astype(jnp.bfloat16)
    W2 = W2.astype(jnp.bfloat16)
    W3 = W3.astype(jnp.bfloat16)
    idx2 = idx.astype(jnp.int32).reshape(B, 1)

    T = 512
    grid = (B // T,)

    def full(shape):
        return pl.BlockSpec(shape, lambda i: (0,) * len(shape))

    ns, rw = pl.pallas_call(
        _fused_kernel,
        grid=grid,
        in_specs=[
            pl.BlockSpec((T, in_dim), lambda i: (i, 0)),
            pl.BlockSpec((T, STATE_DIM), lambda i: (i, 0)),
            pl.BlockSpec((T, 1), lambda i: (i, 0)),
            full((E, in_dim, HIDDEN)),
            full((E, HIDDEN)),
            full((E, HIDDEN, HIDDEN)),
            full((E, HIDDEN)),
            full((E, HIDDEN, OUT_DIM)),
            full((E, OUT_DIM)),
        ],
        out_specs=[
            pl.BlockSpec((T, STATE_DIM), lambda i: (i, 0)),
            pl.BlockSpec((T, 1), lambda i: (i, 0)),
        ],
        out_shape=[
            jax.ShapeDtypeStruct((B, STATE_DIM), jnp.float32),
            jax.ShapeDtypeStruct((B, 1), jnp.float32),
        ],
        compiler_params=pltpu.CompilerParams(
            dimension_semantics=("parallel",)),
    )(x, state, idx2, W1, b1, W2, b2, W3, b3)
    return (ns, rw)


# speedup vs baseline: 2.8549x; 1.2430x over previous
import jax
import jax.numpy as jnp
from jax.experimental import pallas as pl
from jax.experimental.pallas import tpu as pltpu

STATE_DIM = 128
ACTION_DIM = 32
HIDDEN = 256
E = 8
OUT_DIM = STATE_DIM + 1


def _fused_kernel(x_ref, state_ref, idx_ref, W1_ref, b1_ref, W2_ref, b2_ref,
                  W3_ref, b3_ref, ns_ref, rw_ref):
    x = x_ref[:]
    idx = idx_ref[:]
    acc = jnp.zeros((x.shape[0], OUT_DIM), dtype=jnp.float32)
    for e in range(E):
        h1 = jnp.maximum(
            jnp.dot(x, W1_ref[e], preferred_element_type=jnp.float32) + b1_ref[e], 0.0)
        h2 = jnp.maximum(
            jnp.dot(h1.astype(jnp.bfloat16), W2_ref[e],
                    preferred_element_type=jnp.float32) + b2_ref[e], 0.0)
        o = jnp.dot(h2.astype(jnp.bfloat16), W3_ref[e],
                    preferred_element_type=jnp.float32) + b3_ref[e]
        acc = jnp.where(idx == e, o, acc)
    ns_ref[:] = state_ref[:] + acc[:, :STATE_DIM]
    rw_ref[:] = acc[:, STATE_DIM:]


@jax.jit
def kernel(state, action, W1, b1, W2, b2, W3, b3, idx):
    B = state.shape[0]
    in_dim = STATE_DIM + ACTION_DIM
    x = jnp.concatenate([state, action], axis=-1).astype(jnp.bfloat16)
    W1 = W1.astype(jnp.bfloat16)
    W2 = W2.astype(jnp.bfloat16)
    W3 = W3.astype(jnp.bfloat16)
    idx2 = idx.astype(jnp.int32).reshape(B, 1)

    T = 1024
    grid = (B // T,)

    def full(shape):
        return pl.BlockSpec(shape, lambda i: (0,) * len(shape))

    ns, rw = pl.pallas_call(
        _fused_kernel,
        grid=grid,
        in_specs=[
            pl.BlockSpec((T, in_dim), lambda i: (i, 0)),
            pl.BlockSpec((T, STATE_DIM), lambda i: (i, 0)),
            pl.BlockSpec((T, 1), lambda i: (i, 0)),
            full((E, in_dim, HIDDEN)),
            full((E, HIDDEN)),
            full((E, HIDDEN, HIDDEN)),
            full((E, HIDDEN)),
            full((E, HIDDEN, OUT_DIM)),
            full((E, OUT_DIM)),
        ],
        out_specs=[
            pl.BlockSpec((T, STATE_DIM), lambda i: (i, 0)),
            pl.BlockSpec((T, 1), lambda i: (i, 0)),
        ],
        out_shape=[
            jax.ShapeDtypeStruct((B, STATE_DIM), jnp.float32),
            jax.ShapeDtypeStruct((B, 1), jnp.float32),
        ],
        compiler_params=pltpu.CompilerParams(
            dimension_semantics=("parallel",)),
    )(x, state, idx2, W1, b1, W2, b2, W3, b3)
    return (ns, rw)


# speedup vs baseline: 2.9141x; 1.0207x over previous
import jax
import jax.numpy as jnp
from jax.experimental import pallas as pl
from jax.experimental.pallas import tpu as pltpu

STATE_DIM = 128
ACTION_DIM = 32
HIDDEN = 256
E = 8
OUT_DIM = STATE_DIM + 1


def _fused_kernel(x_ref, state_ref, idx_ref, W1_ref, b1_ref, W2_ref, b2_ref,
                  W3_ref, b3_ref, ns_ref, rw_ref):
    x = x_ref[:]
    idx = idx_ref[:]
    acc = jnp.zeros((x.shape[0], OUT_DIM), dtype=jnp.float32)
    for e in range(E):
        h1 = jnp.maximum(
            jnp.dot(x, W1_ref[e], preferred_element_type=jnp.float32) + b1_ref[e], 0.0)
        h2 = jnp.maximum(
            jnp.dot(h1.astype(jnp.bfloat16), W2_ref[e],
                    preferred_element_type=jnp.float32) + b2_ref[e], 0.0)
        o = jnp.dot(h2.astype(jnp.bfloat16), W3_ref[e],
                    preferred_element_type=jnp.float32) + b3_ref[e]
        acc = jnp.where(idx == e, o, acc)
    ns_ref[:] = state_ref[:] + acc[:, :STATE_DIM]
    rw_ref[:] = acc[:, STATE_DIM:]


@jax.jit
def kernel(state, action, W1, b1, W2, b2, W3, b3, idx):
    B = state.shape[0]
    in_dim = STATE_DIM + ACTION_DIM
    x = jnp.concatenate([state, action], axis=-1).astype(jnp.bfloat16)
    W1 = W1.astype(jnp.bfloat16)
    W2 = W2.astype(jnp.bfloat16)
    W3 = W3.astype(jnp.bfloat16)
    idx2 = idx.astype(jnp.int32).reshape(B, 1)

    T = 2048
    grid = (B // T,)

    def full(shape):
        return pl.BlockSpec(shape, lambda i: (0,) * len(shape))

    ns, rw = pl.pallas_call(
        _fused_kernel,
        grid=grid,
        in_specs=[
            pl.BlockSpec((T, in_dim), lambda i: (i, 0)),
            pl.BlockSpec((T, STATE_DIM), lambda i: (i, 0)),
            pl.BlockSpec((T, 1), lambda i: (i, 0)),
            full((E, in_dim, HIDDEN)),
            full((E, HIDDEN)),
            full((E, HIDDEN, HIDDEN)),
            full((E, HIDDEN)),
            full((E, HIDDEN, OUT_DIM)),
            full((E, OUT_DIM)),
        ],
        out_specs=[
            pl.BlockSpec((T, STATE_DIM), lambda i: (i, 0)),
            pl.BlockSpec((T, 1), lambda i: (i, 0)),
        ],
        out_shape=[
            jax.ShapeDtypeStruct((B, STATE_DIM), jnp.float32),
            jax.ShapeDtypeStruct((B, 1), jnp.float32),
        ],
        compiler_params=pltpu.CompilerParams(
            dimension_semantics=("parallel",)),
    )(x, state, idx2, W1, b1, W2, b2, W3, b3)
    return (ns, rw)
